# baseline (speedup 1.0000x reference)
import jax
import jax.numpy as jnp
from jax import lax
from jax.experimental import pallas as pl
from jax.experimental.pallas import tpu as pltpu
from jax.experimental.pallas import tpu_sc as plsc

B = 16384
S = 64
NC = 2
NS = 16
NW = NC * NS
L = 16
PB = B // NW
GROUPS = PB // L


def _sc_body(m_hbm, s_hbm, w_hbm, b_hbm, u_hbm, out_hbm,
             idx_v, wm_v, um_v, s_v, bm_v, sem_w, sem_u, sem_b, sem_s):
    wid = lax.axis_index("s") * NC + lax.axis_index("c")
    base = wid * PB

    pltpu.sync_copy(m_hbm.at[pl.ds(base, PB)], idx_v)
    cw = pltpu.make_async_copy(w_hbm.at[idx_v], wm_v, sem_w)
    cu = pltpu.make_async_copy(u_hbm.at[idx_v], um_v, sem_u)
    cb = pltpu.make_async_copy(b_hbm.at[idx_v], bm_v, sem_b)
    cs = pltpu.make_async_copy(s_hbm.at[pl.ds(base, PB)], s_v, sem_s)
    cw.start()
    cu.start()
    cb.start()
    cs.start()
    cw.wait()
    cs.wait()
    cb.wait()
    cu.wait()

    def group(g, carry):
        rows = jnp.full((L,), g * L, jnp.int32) + lax.iota(jnp.int32, L)
        accs = [jnp.zeros((L,), jnp.float32) for _ in range(4)]
        for d in range(S):
            col = jnp.full((L,), d, jnp.int32)
            wv = plsc.load_gather(wm_v, [rows, col])
            sv = plsc.load_gather(s_v, [rows, col])
            accs[d % 4] = accs[d % 4] + wv * sv
        inner = (accs[0] + accs[1]) + (accs[2] + accs[3])
        inner = inner + plsc.load_gather(bm_v, [rows])
        e = jnp.exp(inner + inner)
        t = 1.0 - 2.0 / (e + 1.0)
        for d in range(S):
            col = jnp.full((L,), d, jnp.int32)
            uv = plsc.load_gather(um_v, [rows, col])
            sv = plsc.load_gather(s_v, [rows, col])
            plsc.store_scatter(wm_v, [rows, col], sv + uv * t)
        return carry

    lax.fori_loop(0, GROUPS, group, 0)
    pltpu.sync_copy(wm_v, out_hbm.at[pl.ds(base, PB)])


def kernel(m, s, w, b, u):
    mesh = plsc.VectorSubcoreMesh(core_axis_name="c", subcore_axis_name="s")
    run = pl.kernel(
        _sc_body,
        out_type=jax.ShapeDtypeStruct((B, S), jnp.float32),
        mesh=mesh,
        compiler_params=pltpu.CompilerParams(
            needs_layout_passes=False, use_tc_tiling_on_sc=False),
        scratch_types=[
            pltpu.VMEM((PB,), jnp.int32),
            pltpu.VMEM((PB, S), jnp.float32),
            pltpu.VMEM((PB, S), jnp.float32),
            pltpu.VMEM((PB, S), jnp.float32),
            pltpu.VMEM((PB,), jnp.float32),
            pltpu.SemaphoreType.DMA,
            pltpu.SemaphoreType.DMA,
            pltpu.SemaphoreType.DMA,
            pltpu.SemaphoreType.DMA,
        ],
    )
    return run(m.astype(jnp.int32), s, w, b, u)


# speedup vs baseline: 1.5332x; 1.5332x over previous
import jax
import jax.numpy as jnp
from jax import lax
from jax.experimental import pallas as pl
from jax.experimental.pallas import tpu as pltpu
from jax.experimental.pallas import tpu_sc as plsc

B = 16384
S = 64
NC = 2
NS = 16
NW = NC * NS
L = 16
PB = B // NW
GROUPS = PB // L


def _sc_body(m_hbm, s_hbm, w_hbm, b_hbm, u_hbm, out_hbm,
             idx_v, wm_v, um_v, s_v, bm_v, sem_w, sem_u, sem_b, sem_s):
    wid = lax.axis_index("s") * NC + lax.axis_index("c")
    base = wid * PB

    pltpu.sync_copy(m_hbm.at[pl.ds(base, PB)], idx_v)
    cw = pltpu.make_async_copy(w_hbm.at[idx_v], wm_v, sem_w)
    cu = pltpu.make_async_copy(u_hbm.at[idx_v], um_v, sem_u)
    cb = pltpu.make_async_copy(b_hbm.at[idx_v], bm_v.at[pl.ds(0, PB)], sem_b)
    cs = pltpu.make_async_copy(s_hbm.at[pl.ds(base, PB)], s_v, sem_s)
    cw.start()
    cu.start()
    cb.start()
    cs.start()
    cw.wait()
    cs.wait()
    cb.wait()
    cu.wait()

    lane15 = jnp.full((L,), 15, jnp.int32)
    lane0 = jnp.zeros((L,), jnp.int32)

    @plsc.parallel_loop(0, PB, 1, unroll=8)
    def _body(i):
        sv = [s_v[i, pl.ds(16 * k, L)] for k in range(S // L)]
        wv = [wm_v[i, pl.ds(16 * k, L)] for k in range(S // L)]
        uv = [um_v[i, pl.ds(16 * k, L)] for k in range(S // L)]
        p = (wv[0] * sv[0] + wv[1] * sv[1]) + (wv[2] * sv[2] + wv[3] * sv[3])
        c = plsc.cumsum(p)
        inner = jnp.take_along_axis(c, lane15, axis=0)
        bvec = jnp.take_along_axis(bm_v[pl.ds(i, L)], lane0, axis=0)
        x = inner + bvec
        t = 1.0 - 2.0 / (jnp.exp(x + x) + 1.0)
        for k in range(S // L):
            wm_v[i, pl.ds(16 * k, L)] = sv[k] + uv[k] * t
    pltpu.sync_copy(wm_v, out_hbm.at[pl.ds(base, PB)])


def kernel(m, s, w, b, u):
    mesh = plsc.VectorSubcoreMesh(core_axis_name="c", subcore_axis_name="s")
    run = pl.kernel(
        _sc_body,
        out_type=jax.ShapeDtypeStruct((B, S), jnp.float32),
        mesh=mesh,
        compiler_params=pltpu.CompilerParams(
            needs_layout_passes=False, use_tc_tiling_on_sc=False),
        scratch_types=[
            pltpu.VMEM((PB,), jnp.int32),
            pltpu.VMEM((PB, S), jnp.float32),
            pltpu.VMEM((PB, S), jnp.float32),
            pltpu.VMEM((PB, S), jnp.float32),
            pltpu.VMEM((PB + L,), jnp.float32),
            pltpu.SemaphoreType.DMA,
            pltpu.SemaphoreType.DMA,
            pltpu.SemaphoreType.DMA,
            pltpu.SemaphoreType.DMA,
        ],
    )
    return run(m.astype(jnp.int32), s, w, b, u)
